# baseline (speedup 1.0000x reference)
import functools

import jax
import jax.numpy as jnp
from jax import lax
from jax.experimental import pallas as pl
from jax.experimental.pallas import tpu as pltpu
from jax.experimental.pallas import tpu_sc as plsc

B = 32768
K = 8192
D = 32
BB = 256
KC = 512
NKC = K // KC
NG = KC // 8
COMMITMENT = 0.25


def _argmin_body(x_ref, cb_ref, idx_ref, loss_ref, bv_ref, bi_ref):
    i = pl.program_id(0)
    k = pl.program_id(1)
    x = x_ref[...]
    a = jnp.sum(x * x, axis=1, keepdims=True)
    cbk = cb_ref[...]
    bk = jnp.sum(cbk * cbk, axis=1)
    m = lax.dot_general(x, cbk, (((1,), (1,)), ((), ())),
                        preferred_element_type=jnp.float32)
    d = (a + bk[None, :]) - 2.0 * m

    @pl.when(k == 0)
    def _():
        bv_ref[...] = jnp.full((BB, 128), jnp.inf, dtype=jnp.float32)
        bi_ref[...] = jnp.zeros((BB, 128), dtype=jnp.int32)

    d0, d1 = d[:, 0:128], d[:, 128:256]
    d2, d3 = d[:, 256:384], d[:, 384:512]
    m01 = jnp.minimum(d0, d1)
    g01 = jnp.where(d1 < d0, jnp.int32(128), jnp.int32(0))
    m23 = jnp.minimum(d2, d3)
    g23 = jnp.where(d3 < d2, jnp.int32(384), jnp.int32(256))
    dmin = jnp.minimum(m01, m23)
    gbase = jnp.where(m23 < m01, g23, g01)

    bv = bv_ref[...]
    upd = dmin < bv
    bv_ref[...] = jnp.minimum(bv, dmin)
    bi_ref[...] = jnp.where(upd, gbase + k * KC, bi_ref[...])

    @pl.when(k == NKC - 1)
    def _():
        bvf = bv_ref[...]
        full_idx = bi_ref[...] + lax.broadcasted_iota(jnp.int32, (BB, 128), 1)
        minv = jnp.min(bvf, axis=1, keepdims=True)
        idxm = jnp.where(bvf == minv, full_idx, jnp.int32(2**31 - 1))
        idx_ref[...] = jnp.min(idxm, axis=1)

        @pl.when(i == 0)
        def _():
            loss_ref[...] = jnp.zeros((1, 1), dtype=jnp.float32)

        loss_ref[...] += jnp.sum(minv).reshape(1, 1)


_dist_argmin = pl.pallas_call(
    _argmin_body,
    grid=(B // BB, NKC),
    in_specs=[
        pl.BlockSpec((BB, D), lambda i, k: (i, 0)),
        pl.BlockSpec((KC, D), lambda i, k: (k, 0)),
    ],
    out_specs=[
        pl.BlockSpec((BB,), lambda i, k: (i,)),
        pl.BlockSpec((1, 1), lambda i, k: (0, 0)),
    ],
    out_shape=[
        jax.ShapeDtypeStruct((B,), jnp.int32),
        jax.ShapeDtypeStruct((1, 1), jnp.float32),
    ],
    scratch_shapes=[
        pltpu.VMEM((BB, 128), jnp.float32),
        pltpu.VMEM((BB, 128), jnp.int32),
    ],
    compiler_params=pltpu.CompilerParams(
        dimension_semantics=("arbitrary", "arbitrary"),
    ),
)


_NW = 32
_NCORES = 2
_BPW = B // _NW
_CH = 128
_NCH = _BPW // _CH


@functools.cache
def _make_gather():
    mesh = plsc.VectorSubcoreMesh(core_axis_name="c", subcore_axis_name="s")

    @functools.partial(
        pl.kernel,
        mesh=mesh,
        out_type=jax.ShapeDtypeStruct((_NW, _NCH, _CH, D), jnp.float32),
        scratch_types=[
            pltpu.VMEM((_NCH, _CH), jnp.int32),
            pltpu.VMEM((_NCH, _CH, D), jnp.float32),
            pltpu.SemaphoreType.DMA,
        ],
        compiler_params=pltpu.CompilerParams(use_tc_tiling_on_sc=False),
    )
    def _gather_body(cb_hbm, idx_hbm, out_hbm, idx_v, rows_v, sem):
        wid = lax.axis_index("s") * _NCORES + lax.axis_index("c")
        pltpu.sync_copy(idx_hbm.at[wid], idx_v)
        copies = [
            pltpu.async_copy(cb_hbm.at[idx_v.at[j]], rows_v.at[j], sem)
            for j in range(_NCH)
        ]
        for cp in copies:
            cp.wait()
        pltpu.sync_copy(rows_v, out_hbm.at[wid])

    return _gather_body


def kernel(inputs, codebook):
    idx, loss_acc = _dist_argmin(inputs, codebook)
    rows = _make_gather()(codebook, idx.reshape(_NW, _NCH, _CH))
    quantized = rows.reshape(B, D)
    mean_sq = loss_acc[0, 0] / (B * D)
    loss = mean_sq + COMMITMENT * mean_sq
    quantized_st = inputs + (quantized - inputs)
    return quantized_st, loss
